# baseline (speedup 1.0000x reference)
import functools

import jax
import jax.numpy as jnp
from jax import lax
from jax.experimental import pallas as pl
from jax.experimental.pallas import tpu as pltpu
from jax.experimental.pallas import tpu_sc as plsc


def kernel(x, tok_emb, pos_emb):
    B, T = x.shape
    V, D = tok_emb.shape
    L = 16

    info = plsc.get_sparse_core_info()
    NC, NS = info.num_cores, info.num_subcores
    NW = NC * NS
    t_span = T // NW
    C = 16
    nch = t_span // C
    G = nch * B
    NBUF = 4
    SPI = 2 * B
    NIT = G // SPI
    VPR = D // L

    mesh = plsc.VectorSubcoreMesh(core_axis_name="c", subcore_axis_name="s")

    @functools.partial(
        pl.kernel,
        mesh=mesh,
        out_type=jax.ShapeDtypeStruct((B * T, D), jnp.float32),
        scratch_types=[
            pltpu.VMEM((B, t_span), jnp.int32),
            pltpu.VMEM((NBUF, C, D), jnp.float32),
            pltpu.VMEM((2, C, D), jnp.float32),
            pltpu.SemaphoreType.DMA,
            pltpu.SemaphoreType.DMA,
            pltpu.SemaphoreType.DMA,
        ],
    )
    def emb_kernel(x_hbm, tok_hbm, pos_hbm, out_hbm, idx_v, rows_v, pos_v,
                   sem_g, sem_o, sem_p):
        wid = lax.axis_index("s") * NC + lax.axis_index("c")
        t0 = wid * t_span

        idx_cp = pltpu.async_copy(x_hbm.at[:, pl.ds(t0, t_span)], idx_v,
                                  sem_g)
        for c in range(2):
            pltpu.async_copy(pos_hbm.at[pl.ds(t0 + c * C, C)], pos_v.at[c],
                             sem_p)
        idx_cp.wait()

        for b in range(2):
            pltpu.async_copy(
                tok_hbm.at[idx_v.at[b, pl.ds(0, C)]], rows_v.at[b], sem_g)

        def drain_gather(slot):
            pltpu.make_async_copy(
                pos_hbm.at[pl.ds(0, C)], rows_v.at[slot], sem_g).wait()

        def drain_out(slot):
            pltpu.make_async_copy(
                pos_hbm.at[pl.ds(0, C)], rows_v.at[slot], sem_o).wait()

        def drain_pos(p):
            pltpu.make_async_copy(
                pos_hbm.at[pl.ds(0, C)], pos_v.at[p], sem_p).wait()

        def loop_body(i, _):
            for k in range(SPI):
                p, b = divmod(k, B)
                ch = 2 * i + p
                g_off = k
                slot = k % NBUF
                if k < 2:
                    @pl.when(i > 0)
                    def _():
                        drain_out((k + 2) % NBUF)
                else:
                    drain_out((k + 2) % NBUF)
                k2 = k + 2
                p2, b2 = divmod(k2 % SPI, B)
                ch2 = 2 * i + (k2 // B)
                if k2 < SPI:
                    pltpu.async_copy(
                        tok_hbm.at[idx_v.at[b2, pl.ds(ch2 * C, C)]],
                        rows_v.at[k2 % NBUF], sem_g)
                else:
                    @pl.when(i < NIT - 1)
                    def _():
                        pltpu.async_copy(
                            tok_hbm.at[idx_v.at[b2, pl.ds(ch2 * C, C)]],
                            rows_v.at[k2 % NBUF], sem_g)
                drain_gather(slot)
                if b == 0:
                    drain_pos(p)

                @plsc.parallel_loop(0, C)
                def add_body(r):
                    for u in range(VPR):
                        plsc.addupdate(
                            rows_v.at[slot, r, pl.ds(u * L, L)],
                            pos_v[p, r, pl.ds(u * L, L)])

                if b == B - 1:
                    @pl.when(ch + 2 < nch)
                    def _():
                        pltpu.async_copy(
                            pos_hbm.at[pl.ds(t0 + (ch + 2) * C, C)],
                            pos_v.at[p], sem_p)
                row0 = b * T + t0 + ch * C
                pltpu.async_copy(
                    rows_v.at[slot], out_hbm.at[pl.ds(row0, C)], sem_o)
            return 0

        lax.fori_loop(0, NIT, loop_body, 0)

        drain_out(2)
        drain_out(3)

    out = emb_kernel(x, tok_emb, pos_emb)
    return out.reshape(B, T, D)


# speedup vs baseline: 1.2846x; 1.2846x over previous
import functools

import jax
import jax.numpy as jnp
from jax import lax
from jax.experimental import pallas as pl
from jax.experimental.pallas import tpu as pltpu
from jax.experimental.pallas import tpu_sc as plsc


def kernel(x, tok_emb, pos_emb):
    B, T = x.shape
    V, D = tok_emb.shape
    L = 16

    info = plsc.get_sparse_core_info()
    NC, NS = info.num_cores, info.num_subcores
    NW = NC * NS
    t_span = T // NW
    C = 16
    nch = t_span // C
    G = nch * B
    NBUF = 4
    SPI = 2 * B
    NIT = G // SPI
    VPR = D // L

    mesh = plsc.VectorSubcoreMesh(core_axis_name="c", subcore_axis_name="s")

    @functools.partial(
        pl.kernel,
        mesh=mesh,
        out_type=jax.ShapeDtypeStruct((B * T, D), jnp.float32),
        scratch_types=[
            pltpu.VMEM((B, t_span), jnp.int32),
            pltpu.VMEM((NBUF, C, D), jnp.float32),
            pltpu.VMEM((2, C, D), jnp.float32),
            pltpu.SemaphoreType.DMA,
            pltpu.SemaphoreType.DMA,
            pltpu.SemaphoreType.DMA,
        ],
    )
    def emb_kernel(x_hbm, tok_hbm, pos_hbm, out_hbm, idx_v, rows_v, pos_v,
                   sem_g, sem_o, sem_p):
        wid = lax.axis_index("s") * NC + lax.axis_index("c")
        t0 = wid * t_span

        idx_cp = pltpu.async_copy(x_hbm.at[:, pl.ds(t0, t_span)], idx_v,
                                  sem_g)
        for c in range(2):
            pltpu.async_copy(pos_hbm.at[pl.ds(t0 + c * C, C)], pos_v.at[c],
                             sem_p)
        idx_cp.wait()

        for b in range(2):
            pltpu.async_copy(
                tok_hbm.at[idx_v.at[b, pl.ds(0, C)]], rows_v.at[b], sem_g)

        def drain_gather(slot):
            pltpu.make_async_copy(
                pos_hbm.at[pl.ds(0, C)], rows_v.at[slot], sem_g).wait()

        def drain_out(slot):
            pltpu.make_async_copy(
                pos_hbm.at[pl.ds(0, C)], rows_v.at[slot], sem_o).wait()

        def drain_pos(p):
            pltpu.make_async_copy(
                pos_hbm.at[pl.ds(0, C)], pos_v.at[p], sem_p).wait()

        def loop_body(i, _):
            for k in range(SPI):
                p, b = divmod(k, B)
                ch = 2 * i + p
                g_off = k
                slot = k % NBUF
                if k < 2:
                    @pl.when(i > 0)
                    def _():
                        drain_out((k + 2) % NBUF)
                else:
                    drain_out((k + 2) % NBUF)
                k2 = k + 2
                p2, b2 = divmod(k2 % SPI, B)
                ch2 = 2 * i + (k2 // B)
                if k2 < SPI:
                    pltpu.async_copy(
                        tok_hbm.at[idx_v.at[b2, pl.ds(ch2 * C, C)]],
                        rows_v.at[k2 % NBUF], sem_g)
                else:
                    @pl.when(i < NIT - 1)
                    def _():
                        pltpu.async_copy(
                            tok_hbm.at[idx_v.at[b2, pl.ds(ch2 * C, C)]],
                            rows_v.at[k2 % NBUF], sem_g)
                drain_gather(slot)
                if b == 0:
                    drain_pos(p)

                @plsc.parallel_loop(0, C * VPR, unroll=8)
                def add_body(j):
                    r = j // VPR
                    col = (j % VPR) * L
                    plsc.addupdate(
                        rows_v.at[slot, r, pl.ds(col, L)],
                        pos_v[p, r, pl.ds(col, L)])

                if b == B - 1:
                    @pl.when(ch + 2 < nch)
                    def _():
                        pltpu.async_copy(
                            pos_hbm.at[pl.ds(t0 + (ch + 2) * C, C)],
                            pos_v.at[p], sem_p)
                row0 = b * T + t0 + ch * C
                pltpu.async_copy(
                    rows_v.at[slot], out_hbm.at[pl.ds(row0, C)], sem_o)
            return 0

        lax.fori_loop(0, NIT, loop_body, 0)

        drain_out(2)
        drain_out(3)

    out = emb_kernel(x, tok_emb, pos_emb)
    return out.reshape(B, T, D)
